# Initial kernel scaffold; baseline (speedup 1.0000x reference)
import jax
import jax.numpy as jnp
from jax.experimental import pallas as pl


def kernel(position_ids, pe):
    raise NotImplementedError("write your pallas kernel here")


# speedup vs baseline: 4.7447x; 4.7447x over previous
import functools

import jax
import jax.numpy as jnp
from jax import lax
from jax.experimental import pallas as pl
from jax.experimental.pallas import tpu as pltpu
from jax.experimental.pallas import tpu_sc as plsc

_MAXP = 15000
_D = 64
_B = 16384
_H = 200
_N = _B * _H
_NC = 2
_NS = 16
_L = 16
_NW = _NC * _NS
_PER_W = _N // _NW
_BC = 512
_NCH = _PER_W // _BC
_GW = 128
_G = _BC // _GW


def _make_kernel():
    mesh = plsc.VectorSubcoreMesh(core_axis_name="c", subcore_axis_name="s")

    @functools.partial(
        pl.kernel,
        mesh=mesh,
        out_type=jax.ShapeDtypeStruct((_N, _D), jnp.float32),
        scratch_types=[
            pltpu.VMEM((_BC,), jnp.int32),
            pltpu.VMEM((_BC, _D), jnp.float32),
            pltpu.SemaphoreType.DMA,
        ],
        compiler_params=pltpu.CompilerParams(use_tc_tiling_on_sc=False),
    )
    def emb(ids_hbm, pe_hbm, out_hbm, idx_v, rows_v, sem):
        wid = lax.axis_index("s") * _NC + lax.axis_index("c")
        base = wid * _PER_W

        def body(ch, carry):
            off = base + ch * _BC
            pltpu.sync_copy(ids_hbm.at[pl.ds(off, _BC)], idx_v)
            for i in range(_BC // _L):
                sl = pl.ds(i * _L, _L)
                idx_v[sl] = jnp.minimum(idx_v[sl], _MAXP - 1)
            cps = [
                pltpu.async_copy(
                    pe_hbm.at[idx_v.at[pl.ds(j * _GW, _GW)]],
                    rows_v.at[pl.ds(j * _GW, _GW)],
                    sem,
                )
                for j in range(_G)
            ]
            for c in cps:
                c.wait()
            pltpu.sync_copy(rows_v, out_hbm.at[pl.ds(off, _BC)])
            return carry

        lax.fori_loop(0, _NCH, body, 0)

    return emb


_emb = _make_kernel()


def kernel(position_ids, pe):
    ids = position_ids.reshape(-1)
    out = _emb(ids, pe)
    return out.reshape(_B, _H, _D)


# speedup vs baseline: 5.1744x; 1.0906x over previous
import functools

import jax
import jax.numpy as jnp
from jax import lax
from jax.experimental import pallas as pl
from jax.experimental.pallas import tpu as pltpu
from jax.experimental.pallas import tpu_sc as plsc

_MAXP = 15000
_D = 64
_B = 16384
_H = 200
_N = _B * _H
_NC = 2
_NS = 16
_L = 16
_NW = _NC * _NS
_PER_W = _N // _NW
_BC = 640
_NCH = _PER_W // _BC
_GW = 128
_G = _BC // _GW
_NG = _NCH // 2


def _make_kernel():
    mesh = plsc.VectorSubcoreMesh(core_axis_name="c", subcore_axis_name="s")

    @functools.partial(
        pl.kernel,
        mesh=mesh,
        out_type=jax.ShapeDtypeStruct((_N, _D), jnp.float32),
        scratch_types=[
            pltpu.VMEM((2, _BC), jnp.int32),
            pltpu.VMEM((2, _BC, _D), jnp.float32),
            pltpu.SemaphoreType.DMA,
            pltpu.SemaphoreType.DMA,
            pltpu.SemaphoreType.DMA,
            pltpu.SemaphoreType.DMA,
        ],
        compiler_params=pltpu.CompilerParams(use_tc_tiling_on_sc=False),
    )
    def emb(ids_hbm, pe_hbm, out_hbm, idx_v, rows_v, gsem, isem, wsem0, wsem1):
        wid = lax.axis_index("s") * _NC + lax.axis_index("c")
        base = wid * _PER_W
        wsems = (wsem0, wsem1)

        def idx_load(ch, b):
            return pltpu.make_async_copy(
                ids_hbm.at[pl.ds(base + ch * _BC, _BC)], idx_v.at[b], isem)

        def out_write(ch, b):
            return pltpu.make_async_copy(
                rows_v.at[b], out_hbm.at[pl.ds(base + ch * _BC, _BC)], wsems[b])

        idx_load(0, 0).start()

        def body(g, carry):
            for b in range(2):
                ch = 2 * g + b
                if b == 0:
                    pl.when(g > 0)(lambda: out_write(2 * g - 2, 0).wait())
                else:
                    pl.when(g > 0)(lambda: out_write(2 * g - 1, 1).wait())
                idx_load(ch, b).wait()
                for i in range(_BC // _L):
                    sl = (b, pl.ds(i * _L, _L))
                    idx_v[sl] = jnp.minimum(idx_v[sl], _MAXP - 1)
                cps = [
                    pltpu.async_copy(
                        pe_hbm.at[idx_v.at[b, pl.ds(j * _GW, _GW)]],
                        rows_v.at[b, pl.ds(j * _GW, _GW)],
                        gsem,
                    )
                    for j in range(_G)
                ]
                if b == 0:
                    idx_load(ch + 1, 1).start()
                else:
                    pl.when(g < _NG - 1)(lambda: idx_load(ch + 1, 0).start())
                for c in cps:
                    c.wait()
                out_write(ch, b).start()
            return carry

        lax.fori_loop(0, _NG, body, 0)
        out_write(_NCH - 2, 0).wait()
        out_write(_NCH - 1, 1).wait()

    return emb


_emb = _make_kernel()


def kernel(position_ids, pe):
    ids = position_ids.reshape(-1)
    out = _emb(ids, pe)
    return out.reshape(_B, _H, _D)
